# baseline (speedup 1.0000x reference)
import functools

import jax
import jax.numpy as jnp
from jax import lax
from jax.experimental import pallas as pl
from jax.experimental.pallas import tpu as pltpu
from jax.experimental.pallas import tpu_sc as plsc

N = 10000
E = 320000
D = 128
G = 64
H = 128
EPS = 1e-5

NC = 2
NS = 16
NW = NC * NS
CHUNK = 128
EPAD = 327680
EROWS = EPAD // CHUNK
TPW = EROWS // NW
TPC = EROWS // NS
SCW = 16
SCN = TPC // SCW
NACC = 10112
ZPW = NACC // NS
NSRC = 10048
FC = 128

_HI = lax.Precision.HIGHEST


def _mm(a, b):
    return lax.dot_general(a, b, (((1,), (0,)), ((), ())),
                           precision=_HI, preferred_element_type=jnp.float32)


def _mmT(a, b):
    return lax.dot_general(a, b, (((0,), (0,)), ((), ())),
                           precision=_HI, preferred_element_type=jnp.float32)


def _mmd(a, b):
    return lax.dot_general(a, b, (((1,), (0,)), ((), ())),
                           preferred_element_type=jnp.float32)


def _leaky(v):
    return jnp.where(v >= 0, v, 0.01 * v)


def _msg_body(xs_hbm, srcr_hbm, dstr_hbm, zeros_hbm, out_hbm,
              sidx, didx, row_a, row_b, acc, sem_a, sem_b, sem_i0, sem_i1):
    s = lax.axis_index("s")
    base = s * TPC
    pltpu.sync_copy(zeros_hbm.at[pl.ds(s * ZPW, ZPW)], acc.at[pl.ds(s * ZPW, ZPW)])

    sems_i = (sem_i0, sem_i1)

    def idx_start(g, b):
        pltpu.async_copy(srcr_hbm.at[pl.ds(base + g * SCW, SCW)], sidx.at[b],
                         sems_i[b])
        pltpu.async_copy(dstr_hbm.at[pl.ds(base + g * SCW, SCW)], didx.at[b],
                         sems_i[b])

    def idx_wait(g, b):
        pltpu.make_async_copy(srcr_hbm.at[pl.ds(base + g * SCW, SCW)],
                              sidx.at[b], sems_i[b]).wait()
        pltpu.make_async_copy(dstr_hbm.at[pl.ds(base + g * SCW, SCW)],
                              didx.at[b], sems_i[b]).wait()

    idx_start(0, 0)
    idx_start(1, 1)
    plsc.subcore_barrier()

    rows = (row_a, row_b)
    sems = (sem_a, sem_b)

    def g_start(b, i, r):
        pltpu.async_copy(xs_hbm.at[sidx.at[b, i]], rows[r], sems[r])

    def g_wait(b, i, r):
        pltpu.make_async_copy(xs_hbm.at[sidx.at[b, i]], rows[r], sems[r]).wait()

    def process(b):
        g_start(b, 0, 0)
        for i in range(SCW):
            if i + 1 < SCW:
                g_start(b, i + 1, (i + 1) % 2)
            g_wait(b, i, i % 2)
            pltpu.sync_copy(rows[i % 2], acc.at[didx.at[b, i]], add=True)

    def body(t, carry):
        g0 = 2 * t
        idx_wait(g0, 0)
        process(0)

        @pl.when(g0 + 2 < SCN)
        def _():
            idx_start(g0 + 2, 0)

        idx_wait(g0 + 1, 1)
        process(1)

        @pl.when(g0 + 3 < SCN)
        def _():
            idx_start(g0 + 3, 1)

        return carry

    lax.fori_loop(0, SCN // 2, body, 0)

    plsc.subcore_barrier()
    pltpu.sync_copy(acc.at[pl.ds(s * ZPW, ZPW)],
                    out_hbm.at[pl.ds(s * ZPW, ZPW)])


def _deg_body(dstr_hbm, zeros_hbm, ones_hbm, out_hbm, didx, ones_v, acc,
              sem_i0, sem_i1):
    s = lax.axis_index("s")
    base = s * TPC
    pltpu.sync_copy(zeros_hbm.at[pl.ds(s * ZPW, ZPW)], acc.at[pl.ds(s * ZPW, ZPW)])
    pltpu.sync_copy(ones_hbm, ones_v)

    sems_i = (sem_i0, sem_i1)

    def idx_start(g, b):
        pltpu.async_copy(dstr_hbm.at[pl.ds(base + g * SCW, SCW)], didx.at[b],
                         sems_i[b])

    def idx_wait(g, b):
        pltpu.make_async_copy(dstr_hbm.at[pl.ds(base + g * SCW, SCW)],
                              didx.at[b], sems_i[b]).wait()

    idx_start(0, 0)
    idx_start(1, 1)
    plsc.subcore_barrier()

    def process(b):
        for i in range(SCW):
            pltpu.sync_copy(ones_v, acc.at[didx.at[b, i]], add=True)

    def body(t, carry):
        g0 = 2 * t
        idx_wait(g0, 0)
        process(0)

        @pl.when(g0 + 2 < SCN)
        def _():
            idx_start(g0 + 2, 0)

        idx_wait(g0 + 1, 1)
        process(1)

        @pl.when(g0 + 3 < SCN)
        def _():
            idx_start(g0 + 3, 1)

        return carry

    lax.fori_loop(0, SCN // 2, body, 0)

    plsc.subcore_barrier()
    pltpu.sync_copy(acc.at[pl.ds(s * ZPW, ZPW)],
                    out_hbm.at[pl.ds(s * ZPW, ZPW)])


@functools.lru_cache(maxsize=None)
def _sc_mesh(num_cores):
    return plsc.VectorSubcoreMesh(core_axis_name="c", subcore_axis_name="s",
                                  num_cores=num_cores, num_subcores=NS)


@functools.lru_cache(maxsize=None)
def _sc_msg_kernel():
    return pl.kernel(
        _msg_body,
        out_type=jax.ShapeDtypeStruct((NACC, H), jnp.float32),
        mesh=_sc_mesh(1),
        scratch_types=[
            pltpu.VMEM((2, SCW, CHUNK), jnp.int32),
            pltpu.VMEM((2, SCW, CHUNK), jnp.int32),
            pltpu.VMEM((CHUNK, H), jnp.float32),
            pltpu.VMEM((CHUNK, H), jnp.float32),
            pltpu.VMEM_SHARED((NACC, H), jnp.float32),
            pltpu.SemaphoreType.DMA,
            pltpu.SemaphoreType.DMA,
            pltpu.SemaphoreType.DMA,
            pltpu.SemaphoreType.DMA,
        ],
    )


def _sc_msg(xs, srcr, dstr, z128):
    return _sc_msg_kernel()(xs, srcr, dstr, z128)


def _sc_msg_emu(xs, srcr, dstr, z128):
    return jnp.zeros((NACC, H), jnp.float32).at[dstr.reshape(-1)].add(
        xs[srcr.reshape(-1)])


def _sc_deg_emu(dstr, z16, ones16):
    acc = jnp.zeros((NACC,), jnp.float32).at[dstr.reshape(-1)].add(1.0)
    return jnp.broadcast_to(acc[:, None], (NACC, 16))


def _sc_deg(dstr, z128, ones128):
    k = pl.kernel(
        _deg_body,
        out_type=jax.ShapeDtypeStruct((NACC, H), jnp.float32),
        mesh=_sc_mesh(1),
        scratch_types=[
            pltpu.VMEM((2, SCW, CHUNK), jnp.int32),
            pltpu.VMEM((CHUNK, H), jnp.float32),
            pltpu.VMEM_SHARED((NACC, H), jnp.float32),
            pltpu.SemaphoreType.DMA,
            pltpu.SemaphoreType.DMA,
        ],
    )
    return k(dstr, z128, ones128)


ROWB = 2000
RB = N // ROWB


def _coef(m, var, w, b, ms):
    rstd = lax.rsqrt(var + EPS)
    a = w * rstd
    return a, b - a * ms * m


def _acc1(i, v, r):
    @pl.when(i == 0)
    def _():
        r[...] = v

    @pl.when(i != 0)
    def _():
        r[...] = r[...] + v


def _acc2(i, va, vb, ra, rb):
    _acc1(i, va, ra)
    _acc1(i, vb, rb)


def _stats1_body(x_ref, pl_ref, st_ref, sts_ref, msx_ref, msp_ref,
                 mx_ref, vx_ref, mp_ref, vp_ref):
    ph = pl.program_id(0)
    i = pl.program_id(1)
    x = x_ref[...]
    p = pl_ref[...]
    sts = sts_ref[...]

    @pl.when(ph == 0)
    def _():
        _acc2(i, _mmT(sts, x), _mmT(sts, p), mx_ref, mp_ref)

    @pl.when(ph == 1)
    def _():
        st = st_ref[...]
        ox = x - msx_ref[...] * _mm(st, mx_ref[...])
        op = p - msp_ref[...] * _mm(st, mp_ref[...])
        _acc2(i, _mmT(sts, ox * ox), _mmT(sts, op * op), vx_ref, vp_ref)


def _apply1_body(x_ref, pl_ref, st_ref, deg_ref, mx_ref, m2x_ref, mp_ref,
                 m2p_ref, gnwx_ref, gnbx_ref, gnmsx_ref, gnwp_ref, gnbp_ref,
                 gnmsp_ref, w1x_ref, w1p_ref, xs_ref, dinv_ref):
    st = st_ref[...]
    ax, bx = _coef(mx_ref[...], m2x_ref[...], gnwx_ref[...], gnbx_ref[...],
                   gnmsx_ref[...])
    ap, bp = _coef(mp_ref[...], m2p_ref[...], gnwp_ref[...], gnbp_ref[...],
                   gnmsp_ref[...])
    gnx = _mm(st, ax) * x_ref[...] + _mm(st, bx)
    gnp = _mm(st, ap) * pl_ref[...] + _mm(st, bp)
    xw = _mmd(gnx, w1x_ref[...]) + gnp * w1p_ref[...]
    dinv = lax.rsqrt(deg_ref[...])
    xs_ref[...] = xw * dinv
    dinv_ref[...] = dinv


def _hstats_body(p_ref, xs_ref, dinv_ref, b_ref, st_ref, sts_ref, ms_ref,
                 h_ref, mh_ref, vh_ref):
    ph = pl.program_id(0)
    i = pl.program_id(1)
    h = _leaky(dinv_ref[...] * (p_ref[...] + xs_ref[...]) + b_ref[...])
    h_ref[...] = h
    sts = sts_ref[...]

    @pl.when(ph == 0)
    def _():
        _acc1(i, _mmT(sts, h), mh_ref)

    @pl.when(ph == 1)
    def _():
        o = h - ms_ref[...] * _mm(st_ref[...], mh_ref[...])
        _acc1(i, _mmT(sts, o * o), vh_ref)


def _apply2_body(h_ref, st_ref, dinv_ref, mh_ref, m2h_ref, gnw_ref, gnb_ref,
                 gnms_ref, w2_ref, xs2_ref):
    st = st_ref[...]
    a, b = _coef(mh_ref[...], m2h_ref[...], gnw_ref[...], gnb_ref[...],
                 gnms_ref[...])
    gn = _mm(st, a) * h_ref[...] + _mm(st, b)
    xs2_ref[...] = _mmd(gn, w2_ref[...]) * dinv_ref[...]


def _bn(v, g, b):
    m = jnp.mean(v, axis=0, keepdims=True)
    var = jnp.mean((v - m) ** 2, axis=0, keepdims=True)
    return g * (v - m) * lax.rsqrt(var + EPS) + b


def _final_body(p_ref, xs2_ref, dinv_ref, b2_ref, h1_ref, sts_ref,
                bn1g_ref, bn1b_ref, fw1_ref, fb1_ref, bn2g_ref, bn2b_ref,
                fw2_ref, fb2_ref, y_ref, pool1_ref, pool2_ref):
    i = pl.program_id(0)
    h2 = _leaky(dinv_ref[...] * (p_ref[...] + xs2_ref[...]) + b2_ref[...])
    sts = sts_ref[...]
    _acc2(i, _mmT(sts, h1_ref[...]), _mmT(sts, h2), pool1_ref, pool2_ref)

    @pl.when(i == RB - 1)
    def _():
        pooled = jnp.concatenate([pool1_ref[...], pool2_ref[...]], axis=1)
        y1 = _leaky(_mmd(_bn(pooled, bn1g_ref[...], bn1b_ref[...]),
                         fw1_ref[...]) + fb1_ref[...])
        y_ref[...] = (_mmd(_bn(y1, bn2g_ref[...], bn2b_ref[...]),
                           fw2_ref[...]) + fb2_ref[...])


def _rblk(cols):
    return pl.BlockSpec((ROWB, cols), lambda i: (i, 0))


def _full(shape):
    return pl.BlockSpec(shape, lambda i: (0, 0))


def _f32(shape):
    return jax.ShapeDtypeStruct(shape, jnp.float32)


def _rblk2(cols):
    return pl.BlockSpec((ROWB, cols), lambda p, r: (r, 0))


def _full2(shape):
    return pl.BlockSpec(shape, lambda p, r: (0, 0))


def _tc_stats1(x, pl2, st, sts, msx, msp):
    return pl.pallas_call(
        _stats1_body,
        grid=(2, RB),
        in_specs=[_rblk2(D), _rblk2(1), _rblk2(G), _rblk2(G),
                  _full2((1, D)), _full2((1, 1))],
        out_specs=[_full2((G, D)), _full2((G, D)),
                   _full2((G, 1)), _full2((G, 1))],
        out_shape=[_f32((G, D)), _f32((G, D)), _f32((G, 1)), _f32((G, 1))],
    )(x, pl2, st, sts, msx, msp)


def _tc_apply1(x, pl2, st, deg, stats, gparams, w1x, w1row):
    return pl.pallas_call(
        _apply1_body,
        grid=(RB,),
        in_specs=[_rblk(D), _rblk(1), _rblk(G), _rblk(1),
                  _full((G, D)), _full((G, D)), _full((G, 1)), _full((G, 1)),
                  _full((1, D)), _full((1, D)), _full((1, D)),
                  _full((1, 1)), _full((1, 1)), _full((1, 1)),
                  _full((D, H)), _full((1, H))],
        out_specs=[_rblk(H), _rblk(1)],
        out_shape=[_f32((N, H)), _f32((N, 1))],
    )(x, pl2, st, deg, *stats, *gparams, w1x, w1row)


def _tc_hstats(p, xs, dinv, br, st, sts, ms):
    return pl.pallas_call(
        _hstats_body,
        grid=(2, RB),
        in_specs=[_rblk2(H), _rblk2(H), _rblk2(1), _full2((1, H)),
                  _rblk2(G), _rblk2(G), _full2((1, H))],
        out_specs=[_rblk2(H), _full2((G, H)), _full2((G, H))],
        out_shape=[_f32((N, H)), _f32((G, H)), _f32((G, H))],
    )(p, xs, dinv, br, st, sts, ms)


def _tc_apply2(h1, st, dinv, mh, m2h, gnw, gnb, gnms, w2):
    return pl.pallas_call(
        _apply2_body,
        grid=(RB,),
        in_specs=[_rblk(H), _rblk(G), _rblk(1),
                  _full((G, H)), _full((G, H)),
                  _full((1, H)), _full((1, H)), _full((1, H)),
                  _full((H, H))],
        out_specs=_rblk(H),
        out_shape=_f32((N, H)),
    )(h1, st, dinv, mh, m2h, gnw, gnb, gnms, w2)


def _tc_final(p2, xs2, dinv, b2r, h1, sts, bn1g, bn1b, fw1, fb1,
              bn2g, bn2b, fw2, fb2):
    return pl.pallas_call(
        _final_body,
        grid=(RB,),
        in_specs=[_rblk(H), _rblk(H), _rblk(1), _full((1, H)), _rblk(H),
                  _rblk(G), _full((1, 2 * H)), _full((1, 2 * H)),
                  _full((2 * H, FC)), _full((1, FC)), _full((1, FC)),
                  _full((1, FC)), _full((FC, 1)), _full((1, 1))],
        out_specs=_full((G, 1)),
        out_shape=_f32((G, 1)),
        scratch_shapes=[pltpu.VMEM((G, H), jnp.float32),
                        pltpu.VMEM((G, H), jnp.float32)],
    )(p2, xs2, dinv, b2r, h1, sts, bn1g, bn1b, fw1, fb1, bn2g, bn2b,
      fw2, fb2)


def _row(v):
    return v.reshape(1, -1).astype(jnp.float32)


def kernel(x, pLDDT, edge_index, batch, gn1_w, gn1_b, gn1_ms, W1, b1,
           gn2_w, gn2_b, gn2_ms, W2, b2, bn1_g, bn1_b, fcW1, fcb1,
           bn2_g, bn2_b, fcW2, fcb2):
    npad = EPAD - E
    pad_src = N + (jnp.arange(npad, dtype=jnp.int32) % (NSRC - N))
    pad_dst = N + (jnp.arange(npad, dtype=jnp.int32) % (NACC - N))
    srcr = jnp.concatenate([edge_index[0], pad_src]).reshape(EROWS, CHUNK)
    dstr = jnp.concatenate([edge_index[1], pad_dst]).reshape(EROWS, CHUNK)

    st = (batch.reshape(N, 1) == jnp.arange(G, dtype=batch.dtype)
          .reshape(1, G)).astype(jnp.float32)
    sts = st / jnp.maximum(jnp.sum(st, axis=0, keepdims=True), 1.0)
    pl2 = pLDDT.reshape(N, 1)

    z128 = jnp.zeros((NACC, H), jnp.float32)
    ones128 = jnp.ones((CHUNK, H), jnp.float32)

    zrows = jnp.zeros((NSRC - N, H), jnp.float32)

    deg2 = _sc_deg(dstr, z128, ones128)
    deg = deg2[:N, 0:1] + 1.0
    stats1 = _tc_stats1(x, pl2, st, sts, _row(gn1_ms[:D]),
                        gn1_ms[D:].reshape(1, 1))
    gparams1 = (_row(gn1_w[:D]), _row(gn1_b[:D]), _row(gn1_ms[:D]),
                gn1_w[D:].reshape(1, 1), gn1_b[D:].reshape(1, 1),
                gn1_ms[D:].reshape(1, 1))
    xs1, dinv = _tc_apply1(x, pl2, st, deg, stats1, gparams1,
                           W1[:D], W1[D:].reshape(1, H))
    p1 = _sc_msg(jnp.concatenate([xs1, zrows], axis=0), srcr, dstr, z128)
    h1, mh, vh = _tc_hstats(p1, xs1, dinv, _row(b1), st, sts, _row(gn2_ms))
    xs2 = _tc_apply2(h1, st, dinv, mh, vh, _row(gn2_w), _row(gn2_b),
                     _row(gn2_ms), W2)
    p2 = _sc_msg(jnp.concatenate([xs2, zrows], axis=0), srcr, dstr, z128)
    y = _tc_final(p2, xs2, dinv, _row(b2), h1, sts, _row(bn1_g), _row(bn1_b),
                  fcW1, _row(fcb1), _row(bn2_g), _row(bn2_b), fcW2, _row(fcb2))
    return y


# speedup vs baseline: 1.0145x; 1.0145x over previous
import functools

import jax
import jax.numpy as jnp
from jax import lax
from jax.experimental import pallas as pl
from jax.experimental.pallas import tpu as pltpu
from jax.experimental.pallas import tpu_sc as plsc

N = 10000
E = 320000
D = 128
G = 64
H = 128
EPS = 1e-5

NC = 2
NS = 16
NW = NC * NS
CHUNK = 128
EPAD = 327680
EROWS = EPAD // CHUNK
TPW = EROWS // NW
TPC = EROWS // NS
SCW = 16
SCN = TPC // SCW
NACC = 10112
ZPW = NACC // NS
FC = 128

_HI = lax.Precision.HIGHEST


def _mm(a, b):
    return lax.dot_general(a, b, (((1,), (0,)), ((), ())),
                           precision=_HI, preferred_element_type=jnp.float32)


def _mmT(a, b):
    return lax.dot_general(a, b, (((0,), (0,)), ((), ())),
                           precision=_HI, preferred_element_type=jnp.float32)


def _mmd(a, b):
    return lax.dot_general(a, b, (((1,), (0,)), ((), ())),
                           preferred_element_type=jnp.float32)


def _leaky(v):
    return jnp.where(v >= 0, v, 0.01 * v)


def _msg_body(xs_hbm, srcr_hbm, dstr_hbm, zeros_hbm, out_hbm,
              sidx, didx, row_a, row_b, acc, sem_a, sem_b, sem_i0, sem_i1):
    s = lax.axis_index("s")
    base = s * TPC
    pltpu.sync_copy(zeros_hbm.at[pl.ds(s * ZPW, ZPW)], acc.at[pl.ds(s * ZPW, ZPW)])

    sems_i = (sem_i0, sem_i1)

    def idx_start(g, b):
        pltpu.async_copy(srcr_hbm.at[pl.ds(base + g * SCW, SCW)], sidx.at[b],
                         sems_i[b])
        pltpu.async_copy(dstr_hbm.at[pl.ds(base + g * SCW, SCW)], didx.at[b],
                         sems_i[b])

    def idx_wait(g, b):
        pltpu.make_async_copy(srcr_hbm.at[pl.ds(base + g * SCW, SCW)],
                              sidx.at[b], sems_i[b]).wait()
        pltpu.make_async_copy(dstr_hbm.at[pl.ds(base + g * SCW, SCW)],
                              didx.at[b], sems_i[b]).wait()

    idx_start(0, 0)
    idx_start(1, 1)
    plsc.subcore_barrier()

    rows = (row_a, row_b)
    sems = (sem_a, sem_b)

    def g_start(b, i, r):
        pltpu.async_copy(xs_hbm.at[sidx.at[b, i]], rows[r], sems[r])

    def g_wait(b, i, r):
        pltpu.make_async_copy(xs_hbm.at[sidx.at[b, i]], rows[r], sems[r]).wait()

    def process(b):
        g_start(b, 0, 0)
        for i in range(SCW):
            if i + 1 < SCW:
                g_start(b, i + 1, (i + 1) % 2)
            g_wait(b, i, i % 2)
            pltpu.sync_copy(rows[i % 2], acc.at[didx.at[b, i]], add=True)

    def body(t, carry):
        g0 = 2 * t
        idx_wait(g0, 0)
        process(0)

        @pl.when(g0 + 2 < SCN)
        def _():
            idx_start(g0 + 2, 0)

        idx_wait(g0 + 1, 1)
        process(1)

        @pl.when(g0 + 3 < SCN)
        def _():
            idx_start(g0 + 3, 1)

        return carry

    lax.fori_loop(0, SCN // 2, body, 0)

    plsc.subcore_barrier()
    pltpu.sync_copy(acc.at[pl.ds(s * ZPW, ZPW)],
                    out_hbm.at[pl.ds(s * ZPW, ZPW)])


def _deg_body(dstr_hbm, zeros_hbm, ones_hbm, out_hbm, didx, ones_v, acc,
              sem_i0, sem_i1):
    s = lax.axis_index("s")
    base = s * TPC
    pltpu.sync_copy(zeros_hbm.at[pl.ds(s * ZPW, ZPW)], acc.at[pl.ds(s * ZPW, ZPW)])
    pltpu.sync_copy(ones_hbm, ones_v)

    sems_i = (sem_i0, sem_i1)

    def idx_start(g, b):
        pltpu.async_copy(dstr_hbm.at[pl.ds(base + g * SCW, SCW)], didx.at[b],
                         sems_i[b])

    def idx_wait(g, b):
        pltpu.make_async_copy(dstr_hbm.at[pl.ds(base + g * SCW, SCW)],
                              didx.at[b], sems_i[b]).wait()

    idx_start(0, 0)
    idx_start(1, 1)
    plsc.subcore_barrier()

    def process(b):
        for i in range(SCW):
            pltpu.sync_copy(ones_v, acc.at[didx.at[b, i]], add=True)

    def body(t, carry):
        g0 = 2 * t
        idx_wait(g0, 0)
        process(0)

        @pl.when(g0 + 2 < SCN)
        def _():
            idx_start(g0 + 2, 0)

        idx_wait(g0 + 1, 1)
        process(1)

        @pl.when(g0 + 3 < SCN)
        def _():
            idx_start(g0 + 3, 1)

        return carry

    lax.fori_loop(0, SCN // 2, body, 0)

    plsc.subcore_barrier()
    pltpu.sync_copy(acc.at[pl.ds(s * ZPW, ZPW)],
                    out_hbm.at[pl.ds(s * ZPW, ZPW)])


@functools.lru_cache(maxsize=None)
def _sc_mesh(num_cores):
    return plsc.VectorSubcoreMesh(core_axis_name="c", subcore_axis_name="s",
                                  num_cores=num_cores, num_subcores=NS)


@functools.lru_cache(maxsize=None)
def _sc_msg_kernel():
    return pl.kernel(
        _msg_body,
        out_type=jax.ShapeDtypeStruct((NACC, H), jnp.float32),
        mesh=_sc_mesh(1),
        scratch_types=[
            pltpu.VMEM((2, SCW, CHUNK), jnp.int32),
            pltpu.VMEM((2, SCW, CHUNK), jnp.int32),
            pltpu.VMEM((CHUNK, H), jnp.float32),
            pltpu.VMEM((CHUNK, H), jnp.float32),
            pltpu.VMEM_SHARED((NACC, H), jnp.float32),
            pltpu.SemaphoreType.DMA,
            pltpu.SemaphoreType.DMA,
            pltpu.SemaphoreType.DMA,
            pltpu.SemaphoreType.DMA,
        ],
    )


def _sc_msg(xs, srcr, dstr, z128):
    return _sc_msg_kernel()(xs, srcr, dstr, z128)


def _sc_msg_emu(xs, srcr, dstr, z128):
    return jnp.zeros((NACC, H), jnp.float32).at[dstr.reshape(-1)].add(
        xs[srcr.reshape(-1)])


def _sc_deg_emu(dstr, z16, ones16):
    acc = jnp.zeros((NACC,), jnp.float32).at[dstr.reshape(-1)].add(1.0)
    return jnp.broadcast_to(acc[:, None], (NACC, 16))


def _sc_deg(dstr, z128, ones128):
    k = pl.kernel(
        _deg_body,
        out_type=jax.ShapeDtypeStruct((NACC, H), jnp.float32),
        mesh=_sc_mesh(1),
        scratch_types=[
            pltpu.VMEM((2, SCW, CHUNK), jnp.int32),
            pltpu.VMEM((CHUNK, H), jnp.float32),
            pltpu.VMEM_SHARED((NACC, H), jnp.float32),
            pltpu.SemaphoreType.DMA,
            pltpu.SemaphoreType.DMA,
        ],
    )
    return k(dstr, z128, ones128)


ROWB = 2000
RB = N // ROWB


def _coef(m, var, w, b, ms):
    rstd = lax.rsqrt(var + EPS)
    a = w * rstd
    return a, b - a * ms * m


def _acc1(i, v, r):
    @pl.when(i == 0)
    def _():
        r[...] = v

    @pl.when(i != 0)
    def _():
        r[...] = r[...] + v


def _acc2(i, va, vb, ra, rb):
    _acc1(i, va, ra)
    _acc1(i, vb, rb)


def _stats1_body(x_ref, pl_ref, st_ref, sts_ref, msx_ref, msp_ref,
                 mx_ref, vx_ref, mp_ref, vp_ref):
    ph = pl.program_id(0)
    i = pl.program_id(1)
    x = x_ref[...]
    p = pl_ref[...]
    sts = sts_ref[...]

    @pl.when(ph == 0)
    def _():
        _acc2(i, _mmT(sts, x), _mmT(sts, p), mx_ref, mp_ref)

    @pl.when(ph == 1)
    def _():
        st = st_ref[...]
        ox = x - msx_ref[...] * _mm(st, mx_ref[...])
        op = p - msp_ref[...] * _mm(st, mp_ref[...])
        _acc2(i, _mmT(sts, ox * ox), _mmT(sts, op * op), vx_ref, vp_ref)


def _apply1_body(x_ref, pl_ref, st_ref, deg_ref, mx_ref, m2x_ref, mp_ref,
                 m2p_ref, gnwx_ref, gnbx_ref, gnmsx_ref, gnwp_ref, gnbp_ref,
                 gnmsp_ref, w1x_ref, w1p_ref, xs_ref, dinv_ref):
    st = st_ref[...]
    ax, bx = _coef(mx_ref[...], m2x_ref[...], gnwx_ref[...], gnbx_ref[...],
                   gnmsx_ref[...])
    ap, bp = _coef(mp_ref[...], m2p_ref[...], gnwp_ref[...], gnbp_ref[...],
                   gnmsp_ref[...])
    gnx = _mm(st, ax) * x_ref[...] + _mm(st, bx)
    gnp = _mm(st, ap) * pl_ref[...] + _mm(st, bp)
    xw = _mmd(gnx, w1x_ref[...]) + gnp * w1p_ref[...]
    dinv = lax.rsqrt(deg_ref[...])
    xs_ref[...] = xw * dinv
    dinv_ref[...] = dinv


def _hstats_body(p_ref, xs_ref, dinv_ref, b_ref, st_ref, sts_ref, ms_ref,
                 h_ref, mh_ref, vh_ref):
    ph = pl.program_id(0)
    i = pl.program_id(1)
    h = _leaky(dinv_ref[...] * (p_ref[...] + xs_ref[...]) + b_ref[...])
    h_ref[...] = h
    sts = sts_ref[...]

    @pl.when(ph == 0)
    def _():
        _acc1(i, _mmT(sts, h), mh_ref)

    @pl.when(ph == 1)
    def _():
        o = h - ms_ref[...] * _mm(st_ref[...], mh_ref[...])
        _acc1(i, _mmT(sts, o * o), vh_ref)


def _apply2_body(h_ref, st_ref, dinv_ref, mh_ref, m2h_ref, gnw_ref, gnb_ref,
                 gnms_ref, w2_ref, xs2_ref):
    st = st_ref[...]
    a, b = _coef(mh_ref[...], m2h_ref[...], gnw_ref[...], gnb_ref[...],
                 gnms_ref[...])
    gn = _mm(st, a) * h_ref[...] + _mm(st, b)
    xs2_ref[...] = _mmd(gn, w2_ref[...]) * dinv_ref[...]


def _bn(v, g, b):
    m = jnp.mean(v, axis=0, keepdims=True)
    var = jnp.mean((v - m) ** 2, axis=0, keepdims=True)
    return g * (v - m) * lax.rsqrt(var + EPS) + b


def _final_body(p_ref, xs2_ref, dinv_ref, b2_ref, h1_ref, sts_ref,
                bn1g_ref, bn1b_ref, fw1_ref, fb1_ref, bn2g_ref, bn2b_ref,
                fw2_ref, fb2_ref, y_ref, pool1_ref, pool2_ref):
    i = pl.program_id(0)
    h2 = _leaky(dinv_ref[...] * (p_ref[...] + xs2_ref[...]) + b2_ref[...])
    sts = sts_ref[...]
    _acc2(i, _mmT(sts, h1_ref[...]), _mmT(sts, h2), pool1_ref, pool2_ref)

    @pl.when(i == RB - 1)
    def _():
        pooled = jnp.concatenate([pool1_ref[...], pool2_ref[...]], axis=1)
        y1 = _leaky(_mmd(_bn(pooled, bn1g_ref[...], bn1b_ref[...]),
                         fw1_ref[...]) + fb1_ref[...])
        y_ref[...] = (_mmd(_bn(y1, bn2g_ref[...], bn2b_ref[...]),
                           fw2_ref[...]) + fb2_ref[...])


def _rblk(cols):
    return pl.BlockSpec((ROWB, cols), lambda i: (i, 0))


def _full(shape):
    return pl.BlockSpec(shape, lambda i: (0, 0))


def _f32(shape):
    return jax.ShapeDtypeStruct(shape, jnp.float32)


def _rblk2(cols):
    return pl.BlockSpec((ROWB, cols), lambda p, r: (r, 0))


def _full2(shape):
    return pl.BlockSpec(shape, lambda p, r: (0, 0))


def _tc_stats1(x, pl2, st, sts, msx, msp):
    return pl.pallas_call(
        _stats1_body,
        grid=(2, RB),
        in_specs=[_rblk2(D), _rblk2(1), _rblk2(G), _rblk2(G),
                  _full2((1, D)), _full2((1, 1))],
        out_specs=[_full2((G, D)), _full2((G, D)),
                   _full2((G, 1)), _full2((G, 1))],
        out_shape=[_f32((G, D)), _f32((G, D)), _f32((G, 1)), _f32((G, 1))],
    )(x, pl2, st, sts, msx, msp)


def _tc_apply1(x, pl2, st, deg, stats, gparams, w1x, w1row):
    return pl.pallas_call(
        _apply1_body,
        grid=(RB,),
        in_specs=[_rblk(D), _rblk(1), _rblk(G), _rblk(1),
                  _full((G, D)), _full((G, D)), _full((G, 1)), _full((G, 1)),
                  _full((1, D)), _full((1, D)), _full((1, D)),
                  _full((1, 1)), _full((1, 1)), _full((1, 1)),
                  _full((D, H)), _full((1, H))],
        out_specs=[_rblk(H), _rblk(1)],
        out_shape=[_f32((N, H)), _f32((N, 1))],
    )(x, pl2, st, deg, *stats, *gparams, w1x, w1row)


def _tc_hstats(p, xs, dinv, br, st, sts, ms):
    return pl.pallas_call(
        _hstats_body,
        grid=(2, RB),
        in_specs=[_rblk2(H), _rblk2(H), _rblk2(1), _full2((1, H)),
                  _rblk2(G), _rblk2(G), _full2((1, H))],
        out_specs=[_rblk2(H), _full2((G, H)), _full2((G, H))],
        out_shape=[_f32((N, H)), _f32((G, H)), _f32((G, H))],
    )(p, xs, dinv, br, st, sts, ms)


def _tc_apply2(h1, st, dinv, mh, m2h, gnw, gnb, gnms, w2):
    return pl.pallas_call(
        _apply2_body,
        grid=(RB,),
        in_specs=[_rblk(H), _rblk(G), _rblk(1),
                  _full((G, H)), _full((G, H)),
                  _full((1, H)), _full((1, H)), _full((1, H)),
                  _full((H, H))],
        out_specs=_rblk(H),
        out_shape=_f32((N, H)),
    )(h1, st, dinv, mh, m2h, gnw, gnb, gnms, w2)


def _tc_final(p2, xs2, dinv, b2r, h1, sts, bn1g, bn1b, fw1, fb1,
              bn2g, bn2b, fw2, fb2):
    return pl.pallas_call(
        _final_body,
        grid=(RB,),
        in_specs=[_rblk(H), _rblk(H), _rblk(1), _full((1, H)), _rblk(H),
                  _rblk(G), _full((1, 2 * H)), _full((1, 2 * H)),
                  _full((2 * H, FC)), _full((1, FC)), _full((1, FC)),
                  _full((1, FC)), _full((FC, 1)), _full((1, 1))],
        out_specs=_full((G, 1)),
        out_shape=_f32((G, 1)),
        scratch_shapes=[pltpu.VMEM((G, H), jnp.float32),
                        pltpu.VMEM((G, H), jnp.float32)],
    )(p2, xs2, dinv, b2r, h1, sts, bn1g, bn1b, fw1, fb1, bn2g, bn2b,
      fw2, fb2)


def _row(v):
    return v.reshape(1, -1).astype(jnp.float32)


def kernel(x, pLDDT, edge_index, batch, gn1_w, gn1_b, gn1_ms, W1, b1,
           gn2_w, gn2_b, gn2_ms, W2, b2, bn1_g, bn1_b, fcW1, fcb1,
           bn2_g, bn2_b, fcW2, fcb2):
    npad = EPAD - E
    pad_src = jnp.arange(npad, dtype=jnp.int32) % 1024
    pad_dst = N + (jnp.arange(npad, dtype=jnp.int32) % (NACC - N))
    srcr = jnp.concatenate([edge_index[0], pad_src]).reshape(EROWS, CHUNK)
    dstr = jnp.concatenate([edge_index[1], pad_dst]).reshape(EROWS, CHUNK)

    st = (batch.reshape(N, 1) == jnp.arange(G, dtype=batch.dtype)
          .reshape(1, G)).astype(jnp.float32)
    sts = st / jnp.maximum(jnp.sum(st, axis=0, keepdims=True), 1.0)
    pl2 = pLDDT.reshape(N, 1)

    z128 = jnp.zeros((NACC, H), jnp.float32)
    ones128 = jnp.ones((CHUNK, H), jnp.float32)

    deg2 = _sc_deg(dstr, z128, ones128)
    deg = deg2[:N, 0:1] + 1.0
    stats1 = _tc_stats1(x, pl2, st, sts, _row(gn1_ms[:D]),
                        gn1_ms[D:].reshape(1, 1))
    gparams1 = (_row(gn1_w[:D]), _row(gn1_b[:D]), _row(gn1_ms[:D]),
                gn1_w[D:].reshape(1, 1), gn1_b[D:].reshape(1, 1),
                gn1_ms[D:].reshape(1, 1))
    xs1, dinv = _tc_apply1(x, pl2, st, deg, stats1, gparams1,
                           W1[:D], W1[D:].reshape(1, H))
    p1 = _sc_msg(xs1, srcr, dstr, z128)
    h1, mh, vh = _tc_hstats(p1, xs1, dinv, _row(b1), st, sts, _row(gn2_ms))
    xs2 = _tc_apply2(h1, st, dinv, mh, vh, _row(gn2_w), _row(gn2_b),
                     _row(gn2_ms), W2)
    p2 = _sc_msg(xs2, srcr, dstr, z128)
    y = _tc_final(p2, xs2, dinv, _row(b2), h1, sts, _row(bn1_g), _row(bn1_b),
                  fcW1, _row(fcb1), _row(bn2_g), _row(bn2_b), fcW2, _row(fcb2))
    return y


# speedup vs baseline: 1.0154x; 1.0010x over previous
import functools

import jax
import jax.numpy as jnp
from jax import lax
from jax.experimental import pallas as pl
from jax.experimental.pallas import tpu as pltpu
from jax.experimental.pallas import tpu_sc as plsc

N = 10000
E = 320000
D = 128
G = 64
H = 128
EPS = 1e-5

NC = 2
NS = 16
NW = NC * NS
CHUNK = 128
EPAD = 327680
EROWS = EPAD // CHUNK
TPC = EROWS // NS
SCW = 16
SCN = TPC // SCW
NACC = 10112
ZPW = NACC // NS
FC = 128

_HI = lax.Precision.HIGHEST


def _mm(a, b):
    return lax.dot_general(a, b, (((1,), (0,)), ((), ())),
                           precision=_HI, preferred_element_type=jnp.float32)


def _mmT(a, b):
    return lax.dot_general(a, b, (((0,), (0,)), ((), ())),
                           precision=_HI, preferred_element_type=jnp.float32)


def _mmd(a, b):
    return lax.dot_general(a, b, (((1,), (0,)), ((), ())),
                           preferred_element_type=jnp.float32)


def _leaky(v):
    return jnp.where(v >= 0, v, 0.01 * v)


def _msg_body(xs_hbm, srcr_hbm, dstr_hbm, zeros_hbm, out_hbm,
              sidx, didx, row_a, row_b, acc, sem_a, sem_b, sem_i0, sem_i1):
    s = lax.axis_index("s")
    base = s * TPC
    pltpu.sync_copy(zeros_hbm.at[pl.ds(s * ZPW, ZPW)], acc.at[pl.ds(s * ZPW, ZPW)])

    sems_i = (sem_i0, sem_i1)

    def idx_start(g, b):
        pltpu.async_copy(srcr_hbm.at[pl.ds(base + g * SCW, SCW)], sidx.at[b],
                         sems_i[b])
        pltpu.async_copy(dstr_hbm.at[pl.ds(base + g * SCW, SCW)], didx.at[b],
                         sems_i[b])

    def idx_wait(g, b):
        pltpu.make_async_copy(srcr_hbm.at[pl.ds(base + g * SCW, SCW)],
                              sidx.at[b], sems_i[b]).wait()
        pltpu.make_async_copy(dstr_hbm.at[pl.ds(base + g * SCW, SCW)],
                              didx.at[b], sems_i[b]).wait()

    idx_start(0, 0)
    idx_start(1, 1)
    plsc.subcore_barrier()

    rows = (row_a, row_b)
    sems = (sem_a, sem_b)

    def g_start(b, i, r):
        pltpu.async_copy(xs_hbm.at[sidx.at[b, i]], rows[r], sems[r])

    def g_wait(b, i, r):
        pltpu.make_async_copy(xs_hbm.at[sidx.at[b, i]], rows[r], sems[r]).wait()

    def process(b):
        g_start(b, 0, 0)
        for i in range(SCW):
            if i + 1 < SCW:
                g_start(b, i + 1, (i + 1) % 2)
            g_wait(b, i, i % 2)
            pltpu.sync_copy(rows[i % 2], acc.at[didx.at[b, i]], add=True)

    def body(t, carry):
        g0 = 2 * t
        idx_wait(g0, 0)
        process(0)

        @pl.when(g0 + 2 < SCN)
        def _():
            idx_start(g0 + 2, 0)

        idx_wait(g0 + 1, 1)
        process(1)

        @pl.when(g0 + 3 < SCN)
        def _():
            idx_start(g0 + 3, 1)

        return carry

    lax.fori_loop(0, SCN // 2, body, 0)

    plsc.subcore_barrier()
    pltpu.sync_copy(acc.at[pl.ds(s * ZPW, ZPW)],
                    out_hbm.at[pl.ds(s * ZPW, ZPW)])


def _deg_body(dstr_hbm, zeros_hbm, ones_hbm, out_hbm, didx, ones_v, acc,
              sem_i0, sem_i1):
    s = lax.axis_index("s")
    base = s * TPC
    pltpu.sync_copy(zeros_hbm.at[pl.ds(s * ZPW, ZPW)], acc.at[pl.ds(s * ZPW, ZPW)])
    pltpu.sync_copy(ones_hbm, ones_v)

    sems_i = (sem_i0, sem_i1)

    def idx_start(g, b):
        pltpu.async_copy(dstr_hbm.at[pl.ds(base + g * SCW, SCW)], didx.at[b],
                         sems_i[b])

    def idx_wait(g, b):
        pltpu.make_async_copy(dstr_hbm.at[pl.ds(base + g * SCW, SCW)],
                              didx.at[b], sems_i[b]).wait()

    idx_start(0, 0)
    idx_start(1, 1)
    plsc.subcore_barrier()

    def process(b):
        for i in range(SCW):
            pltpu.sync_copy(ones_v, acc.at[didx.at[b, i]], add=True)

    def body(t, carry):
        g0 = 2 * t
        idx_wait(g0, 0)
        process(0)

        @pl.when(g0 + 2 < SCN)
        def _():
            idx_start(g0 + 2, 0)

        idx_wait(g0 + 1, 1)
        process(1)

        @pl.when(g0 + 3 < SCN)
        def _():
            idx_start(g0 + 3, 1)

        return carry

    lax.fori_loop(0, SCN // 2, body, 0)

    plsc.subcore_barrier()
    pltpu.sync_copy(acc.at[pl.ds(s * ZPW, ZPW)],
                    out_hbm.at[pl.ds(s * ZPW, ZPW)])


@functools.lru_cache(maxsize=None)
def _sc_mesh(num_cores):
    return plsc.VectorSubcoreMesh(core_axis_name="c", subcore_axis_name="s",
                                  num_cores=num_cores, num_subcores=NS)


@functools.lru_cache(maxsize=None)
def _sc_msg_kernel():
    return pl.kernel(
        _msg_body,
        out_type=jax.ShapeDtypeStruct((NACC, H), jnp.float32),
        mesh=_sc_mesh(1),
        scratch_types=[
            pltpu.VMEM((2, SCW, CHUNK), jnp.int32),
            pltpu.VMEM((2, SCW, CHUNK), jnp.int32),
            pltpu.VMEM((CHUNK, H), jnp.float32),
            pltpu.VMEM((CHUNK, H), jnp.float32),
            pltpu.VMEM_SHARED((NACC, H), jnp.float32),
            pltpu.SemaphoreType.DMA,
            pltpu.SemaphoreType.DMA,
            pltpu.SemaphoreType.DMA,
            pltpu.SemaphoreType.DMA,
        ],
    )


def _sc_msg(xs, srcr, dstr, z128):
    return _sc_msg_kernel()(xs, srcr, dstr, z128)


def _sc_deg(dstr, z128, ones128):
    k = pl.kernel(
        _deg_body,
        out_type=jax.ShapeDtypeStruct((NACC, H), jnp.float32),
        mesh=_sc_mesh(1),
        scratch_types=[
            pltpu.VMEM((2, SCW, CHUNK), jnp.int32),
            pltpu.VMEM((CHUNK, H), jnp.float32),
            pltpu.VMEM_SHARED((NACC, H), jnp.float32),
            pltpu.SemaphoreType.DMA,
            pltpu.SemaphoreType.DMA,
        ],
    )
    return k(dstr, z128, ones128)


ROWB = 2000
RB = N // ROWB


def _coef(m, var, w, b, ms):
    rstd = lax.rsqrt(var + EPS)
    a = w * rstd
    return a, b - a * ms * m


def _acc1(i, v, r):
    @pl.when(i == 0)
    def _():
        r[...] = v

    @pl.when(i != 0)
    def _():
        r[...] = r[...] + v


def _acc2(i, va, vb, ra, rb):
    _acc1(i, va, ra)
    _acc1(i, vb, rb)


def _stats1_body(x_ref, pl_ref, st_ref, sts_ref, msx_ref, msp_ref,
                 mx_ref, vx_ref, mp_ref, vp_ref):
    ph = pl.program_id(0)
    i = pl.program_id(1)
    x = x_ref[...]
    p = pl_ref[...]
    sts = sts_ref[...]

    @pl.when(ph == 0)
    def _():
        _acc2(i, _mmT(sts, x), _mmT(sts, p), mx_ref, mp_ref)

    @pl.when(ph == 1)
    def _():
        st = st_ref[...]
        ox = x - msx_ref[...] * _mm(st, mx_ref[...])
        op = p - msp_ref[...] * _mm(st, mp_ref[...])
        _acc2(i, _mmT(sts, ox * ox), _mmT(sts, op * op), vx_ref, vp_ref)


def _apply1_body(x_ref, pl_ref, st_ref, deg_ref, mx_ref, m2x_ref, mp_ref,
                 m2p_ref, gnwx_ref, gnbx_ref, gnmsx_ref, gnwp_ref, gnbp_ref,
                 gnmsp_ref, w1x_ref, w1p_ref, xs_ref, dinv_ref):
    st = st_ref[...]
    ax, bx = _coef(mx_ref[...], m2x_ref[...], gnwx_ref[...], gnbx_ref[...],
                   gnmsx_ref[...])
    ap, bp = _coef(mp_ref[...], m2p_ref[...], gnwp_ref[...], gnbp_ref[...],
                   gnmsp_ref[...])
    gnx = _mm(st, ax) * x_ref[...] + _mm(st, bx)
    gnp = _mm(st, ap) * pl_ref[...] + _mm(st, bp)
    xw = _mmd(gnx, w1x_ref[...]) + gnp * w1p_ref[...]
    dinv = lax.rsqrt(deg_ref[...])
    xs_ref[...] = xw * dinv
    dinv_ref[...] = dinv


def _hstats_body(p_ref, xs_ref, dinv_ref, b_ref, st_ref, sts_ref, ms_ref,
                 h_ref, mh_ref, vh_ref):
    ph = pl.program_id(0)
    i = pl.program_id(1)
    h = _leaky(dinv_ref[...] * (p_ref[...] + xs_ref[...]) + b_ref[...])
    h_ref[...] = h
    sts = sts_ref[...]

    @pl.when(ph == 0)
    def _():
        _acc1(i, _mmT(sts, h), mh_ref)

    @pl.when(ph == 1)
    def _():
        o = h - ms_ref[...] * _mm(st_ref[...], mh_ref[...])
        _acc1(i, _mmT(sts, o * o), vh_ref)


def _apply2_body(h_ref, st_ref, dinv_ref, mh_ref, m2h_ref, gnw_ref, gnb_ref,
                 gnms_ref, w2_ref, xs2_ref):
    st = st_ref[...]
    a, b = _coef(mh_ref[...], m2h_ref[...], gnw_ref[...], gnb_ref[...],
                 gnms_ref[...])
    gn = _mm(st, a) * h_ref[...] + _mm(st, b)
    xs2_ref[...] = _mmd(gn, w2_ref[...]) * dinv_ref[...]


def _bn(v, g, b):
    m = jnp.mean(v, axis=0, keepdims=True)
    var = jnp.mean((v - m) ** 2, axis=0, keepdims=True)
    return g * (v - m) * lax.rsqrt(var + EPS) + b


def _final_body(p_ref, xs2_ref, dinv_ref, b2_ref, h1_ref, sts_ref,
                bn1g_ref, bn1b_ref, fw1_ref, fb1_ref, bn2g_ref, bn2b_ref,
                fw2_ref, fb2_ref, y_ref, pool1_ref, pool2_ref):
    i = pl.program_id(0)
    h2 = _leaky(dinv_ref[...] * (p_ref[...] + xs2_ref[...]) + b2_ref[...])
    sts = sts_ref[...]
    _acc2(i, _mmT(sts, h1_ref[...]), _mmT(sts, h2), pool1_ref, pool2_ref)

    @pl.when(i == RB - 1)
    def _():
        pooled = jnp.concatenate([pool1_ref[...], pool2_ref[...]], axis=1)
        y1 = _leaky(_mmd(_bn(pooled, bn1g_ref[...], bn1b_ref[...]),
                         fw1_ref[...]) + fb1_ref[...])
        y_ref[...] = (_mmd(_bn(y1, bn2g_ref[...], bn2b_ref[...]),
                           fw2_ref[...]) + fb2_ref[...])


def _rblk(cols):
    return pl.BlockSpec((ROWB, cols), lambda i: (i, 0))


def _full(shape):
    return pl.BlockSpec(shape, lambda i: (0, 0))


def _f32(shape):
    return jax.ShapeDtypeStruct(shape, jnp.float32)


def _rblk2(cols):
    return pl.BlockSpec((ROWB, cols), lambda p, r: (r, 0))


def _full2(shape):
    return pl.BlockSpec(shape, lambda p, r: (0, 0))


def _tc_stats1(x, pl2, st, sts, msx, msp):
    return pl.pallas_call(
        _stats1_body,
        grid=(2, RB),
        in_specs=[_rblk2(D), _rblk2(1), _rblk2(G), _rblk2(G),
                  _full2((1, D)), _full2((1, 1))],
        out_specs=[_full2((G, D)), _full2((G, D)),
                   _full2((G, 1)), _full2((G, 1))],
        out_shape=[_f32((G, D)), _f32((G, D)), _f32((G, 1)), _f32((G, 1))],
    )(x, pl2, st, sts, msx, msp)


def _tc_apply1(x, pl2, st, deg, stats, gparams, w1x, w1row):
    return pl.pallas_call(
        _apply1_body,
        grid=(RB,),
        in_specs=[_rblk(D), _rblk(1), _rblk(G), _rblk(1),
                  _full((G, D)), _full((G, D)), _full((G, 1)), _full((G, 1)),
                  _full((1, D)), _full((1, D)), _full((1, D)),
                  _full((1, 1)), _full((1, 1)), _full((1, 1)),
                  _full((D, H)), _full((1, H))],
        out_specs=[_rblk(H), _rblk(1)],
        out_shape=[_f32((N, H)), _f32((N, 1))],
    )(x, pl2, st, deg, *stats, *gparams, w1x, w1row)


def _tc_hstats(p, xs, dinv, br, st, sts, ms):
    return pl.pallas_call(
        _hstats_body,
        grid=(2, RB),
        in_specs=[_rblk2(H), _rblk2(H), _rblk2(1), _full2((1, H)),
                  _rblk2(G), _rblk2(G), _full2((1, H))],
        out_specs=[_rblk2(H), _full2((G, H)), _full2((G, H))],
        out_shape=[_f32((N, H)), _f32((G, H)), _f32((G, H))],
    )(p, xs, dinv, br, st, sts, ms)


def _tc_apply2(h1, st, dinv, mh, m2h, gnw, gnb, gnms, w2):
    return pl.pallas_call(
        _apply2_body,
        grid=(RB,),
        in_specs=[_rblk(H), _rblk(G), _rblk(1),
                  _full((G, H)), _full((G, H)),
                  _full((1, H)), _full((1, H)), _full((1, H)),
                  _full((H, H))],
        out_specs=_rblk(H),
        out_shape=_f32((N, H)),
    )(h1, st, dinv, mh, m2h, gnw, gnb, gnms, w2)


def _tc_final(p2, xs2, dinv, b2r, h1, sts, bn1g, bn1b, fw1, fb1,
              bn2g, bn2b, fw2, fb2):
    return pl.pallas_call(
        _final_body,
        grid=(RB,),
        in_specs=[_rblk(H), _rblk(H), _rblk(1), _full((1, H)), _rblk(H),
                  _rblk(G), _full((1, 2 * H)), _full((1, 2 * H)),
                  _full((2 * H, FC)), _full((1, FC)), _full((1, FC)),
                  _full((1, FC)), _full((FC, 1)), _full((1, 1))],
        out_specs=_full((G, 1)),
        out_shape=_f32((G, 1)),
        scratch_shapes=[pltpu.VMEM((G, H), jnp.float32),
                        pltpu.VMEM((G, H), jnp.float32)],
    )(p2, xs2, dinv, b2r, h1, sts, bn1g, bn1b, fw1, fb1, bn2g, bn2b,
      fw2, fb2)


def _row(v):
    return v.reshape(1, -1).astype(jnp.float32)


def kernel(x, pLDDT, edge_index, batch, gn1_w, gn1_b, gn1_ms, W1, b1,
           gn2_w, gn2_b, gn2_ms, W2, b2, bn1_g, bn1_b, fcW1, fcb1,
           bn2_g, bn2_b, fcW2, fcb2):
    npad = EPAD - E
    pad_src = jnp.arange(npad, dtype=jnp.int32) % 1024
    pad_dst = N + (jnp.arange(npad, dtype=jnp.int32) % (NACC - N))
    srcr = jnp.concatenate([edge_index[0], pad_src]).reshape(EROWS, CHUNK)
    dstr = jnp.concatenate([edge_index[1], pad_dst]).reshape(EROWS, CHUNK)

    st = (batch.reshape(N, 1) == jnp.arange(G, dtype=batch.dtype)
          .reshape(1, G)).astype(jnp.float32)
    sts = st / jnp.maximum(jnp.sum(st, axis=0, keepdims=True), 1.0)
    pl2 = pLDDT.reshape(N, 1)

    z128 = jnp.zeros((NACC, H), jnp.float32)
    ones128 = jnp.ones((CHUNK, H), jnp.float32)

    deg2 = _sc_deg(dstr, z128, ones128)
    deg = deg2[:N, 0:1] + 1.0
    stats1 = _tc_stats1(x, pl2, st, sts, _row(gn1_ms[:D]),
                        gn1_ms[D:].reshape(1, 1))
    gparams1 = (_row(gn1_w[:D]), _row(gn1_b[:D]), _row(gn1_ms[:D]),
                gn1_w[D:].reshape(1, 1), gn1_b[D:].reshape(1, 1),
                gn1_ms[D:].reshape(1, 1))
    xs1, dinv = _tc_apply1(x, pl2, st, deg, stats1, gparams1,
                           W1[:D], W1[D:].reshape(1, H))
    p1 = _sc_msg(xs1, srcr, dstr, z128)
    h1, mh, vh = _tc_hstats(p1, xs1, dinv, _row(b1), st, sts, _row(gn2_ms))
    xs2 = _tc_apply2(h1, st, dinv, mh, vh, _row(gn2_w), _row(gn2_b),
                     _row(gn2_ms), W2)
    p2 = _sc_msg(xs2, srcr, dstr, z128)
    y = _tc_final(p2, xs2, dinv, _row(b2), h1, sts, _row(bn1_g), _row(bn1_b),
                  fcW1, _row(fcb1), _row(bn2_g), _row(bn2_b), fcW2, _row(fcb2))
    return y
